# Initial kernel scaffold; baseline (speedup 1.0000x reference)
import jax
import jax.numpy as jnp
from jax.experimental import pallas as pl


def kernel(edge_index, adj_vals, users, items, emb_user, emb_item):
    raise NotImplementedError("write your pallas kernel here")


# speedup vs baseline: 2.0161x; 2.0161x over previous
import functools

import jax
import jax.numpy as jnp
from jax import lax
from jax.experimental import pallas as pl
from jax.experimental.pallas import tpu as pltpu, tpu_sc as plsc

N_USER = 20000
N_ITEM = 30000
N = N_USER + N_ITEM
E = 800000
D = 64
B = 4096

NC = 2
NS = 16
L = 16

HALF = N // 2
HPAD = 25088
STRIPE = HPAD // NS
NP = 2 * HPAD
SHIFT = HPAD - HALF
DUMP = HALF + 8

K = 128
EPT = 50176
NB = EPT // K
E_PAD = NS * EPT
ZROWS = 224

_mesh = plsc.VectorSubcoreMesh(core_axis_name="c", subcore_axis_name="s")


def _layer_body(row_hbm, col_hbm, vals_hbm, xprev, out,
                col_v, row_v, sidx_v, vals_v, rows_v, zbuf, acc, sem):
    c = lax.axis_index("c")
    s = lax.axis_index("s")
    lane = lax.iota(jnp.int32, L)

    def zero_row(i, _):
        for j in range(D // L):
            zbuf[i, pl.ds(j * L, L)] = jnp.zeros((L,), jnp.float32)
        return 0

    lax.fori_loop(0, ZROWS, zero_row, 0)
    for i in range(STRIPE // ZROWS):
        pltpu.sync_copy(zbuf, acc.at[pl.ds(s * STRIPE + i * ZROWS, ZROWS)])
    plsc.subcore_barrier()

    half_base = c * HALF

    def block(nb, _):
        off = s * EPT + nb * K
        pltpu.sync_copy(row_hbm.at[pl.ds(off, K)], row_v)
        pltpu.sync_copy(col_hbm.at[pl.ds(off, K)], col_v)
        pltpu.sync_copy(vals_hbm.at[pl.ds(off, K)], vals_v)

        for g in range(K // L):
            sl = pl.ds(g * L, L)
            r16 = row_v[sl]
            t = r16 - half_base
            m = (t >= 0) & (t < HALF)
            dump = DUMP + ((lane + g * L) & 63)
            sidx_v[sl] = jnp.where(m, t, dump)
            c16 = col_v[sl]
            col_v[sl] = c16 + jnp.where(c16 >= HALF, SHIFT, 0)

        pltpu.async_copy(xprev.at[col_v], rows_v, sem).wait()

        def scale(g, _):
            v16 = vals_v[pl.ds(g * L, L)]
            for i in range(L):
                v = v16[i]
                e = g * L + i
                for j in range(D // L):
                    sl = pl.ds(j * L, L)
                    rows_v[e, sl] = rows_v[e, sl] * v
            return 0

        lax.fori_loop(0, K // L, scale, 0)
        pltpu.sync_copy(rows_v, acc.at[sidx_v], add=True)
        return 0

    lax.fori_loop(0, NB, block, 0)
    plsc.subcore_barrier()

    for i in range(STRIPE // ZROWS):
        o = s * STRIPE + i * ZROWS
        pltpu.sync_copy(acc.at[pl.ds(o, ZROWS)],
                        out.at[pl.ds(c * HPAD + o, ZROWS)])


_params = pltpu.CompilerParams(
    use_tc_tiling_on_sc=False, needs_layout_passes=False)

_layer = functools.partial(
    pl.kernel,
    out_type=jax.ShapeDtypeStruct((NP, D), jnp.float32),
    mesh=_mesh,
    compiler_params=_params,
    scratch_types=[
        pltpu.VMEM((K,), jnp.int32),
        pltpu.VMEM((K,), jnp.int32),
        pltpu.VMEM((K,), jnp.int32),
        pltpu.VMEM((K,), jnp.float32),
        pltpu.VMEM((K, D), jnp.float32),
        pltpu.VMEM((ZROWS, D), jnp.float32),
        pltpu.VMEM_SHARED((HPAD, D), jnp.float32),
        pltpu.SemaphoreType.DMA,
    ],
)(_layer_body)

BPT = B // (NC * NS)


def _gamma_body(x0, x1, x2, x3, users, items, out,
                uidx, iidx, tmp, usum, isum, gout, sem):
    c = lax.axis_index("c")
    s = lax.axis_index("s")
    base = (s * NC + c) * BPT

    pltpu.sync_copy(users.at[pl.ds(base, BPT)], uidx)
    pltpu.sync_copy(items.at[pl.ds(base, BPT)], iidx)

    for g in range(BPT // L):
        sl = pl.ds(g * L, L)
        iv = iidx[sl] + N_USER
        iidx[sl] = iv + jnp.where(iv >= HALF, SHIFT, 0)

    def accumulate(idx, dst):
        pltpu.async_copy(x0.at[idx], dst, sem).wait()
        for tab in (x1, x2, x3):
            pltpu.async_copy(tab.at[idx], tmp, sem).wait()

            def add_row(r, _):
                for j in range(D // L):
                    sl = pl.ds(j * L, L)
                    dst[r, sl] = dst[r, sl] + tmp[r, sl]
                return 0

            lax.fori_loop(0, BPT, add_row, 0)

    accumulate(uidx, usum)
    accumulate(iidx, isum)

    lane = lax.iota(jnp.int32, L)

    def dot_group(g, _):
        r16 = g * L + lane
        acc = jnp.zeros((L,), jnp.float32)
        for d in range(D):
            cidx = jnp.full((L,), d, jnp.int32)
            u = plsc.load_gather(usum, [r16, cidx])
            v = plsc.load_gather(isum, [r16, cidx])
            acc = acc + u * v
        gout[pl.ds(g * L, L)] = acc * jnp.float32(1.0 / 16.0)
        return 0

    lax.fori_loop(0, BPT // L, dot_group, 0)
    pltpu.sync_copy(gout, out.at[pl.ds(base, BPT)])


_gamma = functools.partial(
    pl.kernel,
    out_type=jax.ShapeDtypeStruct((B,), jnp.float32),
    mesh=_mesh,
    compiler_params=_params,
    scratch_types=[
        pltpu.VMEM((BPT,), jnp.int32),
        pltpu.VMEM((BPT,), jnp.int32),
        pltpu.VMEM((BPT, D), jnp.float32),
        pltpu.VMEM((BPT, D), jnp.float32),
        pltpu.VMEM((BPT, D), jnp.float32),
        pltpu.VMEM((BPT,), jnp.float32),
        pltpu.SemaphoreType.DMA,
    ],
)(_gamma_body)


def kernel(edge_index, adj_vals, users, items, emb_user, emb_item):
    row = edge_index[0]
    col = edge_index[1]
    pad = E_PAD - E
    row_p = jnp.concatenate([row, jnp.full((pad,), N, jnp.int32)])
    col_p = jnp.concatenate([col, jnp.zeros((pad,), jnp.int32)])
    vals_p = jnp.concatenate([adj_vals, jnp.zeros((pad,), jnp.float32)])

    zrow = jnp.zeros((SHIFT, D), jnp.float32)
    xp0 = jnp.concatenate(
        [emb_user, emb_item[: HALF - N_USER], zrow,
         emb_item[HALF - N_USER:], zrow], axis=0)

    xp1 = _layer(row_p, col_p, vals_p, xp0)
    xp2 = _layer(row_p, col_p, vals_p, xp1)
    xp3 = _layer(row_p, col_p, vals_p, xp2)
    return _gamma(xp0, xp1, xp2, xp3, users, items)
